# baseline (speedup 1.0000x reference)
import jax
import jax.numpy as jnp
from jax import lax
from jax.experimental import pallas as pl
from jax.experimental.pallas import tpu as pltpu
from jax.experimental.pallas import tpu_sc as plsc

V = 1000000
D = 64
HID = 600
OUT = 1000
B = 4096
TLEN = 20
SLEN = 200

NC = 2
NS = 16
NW = NC * NS
BPW = B // NW
NLANE = 16
NV = D // NLANE

_SKIP_COMPUTE = False
CW = 256
NBF = V // CW
VTAIL = V - NBF * CW
NSLOT = (NBF + NW - 1) // NW + 1


def _untile_compute(blk, outst, width):
    iota = lax.iota(jnp.int32, NLANE)

    def body(i16, carry):
        base = i16 * NLANE
        for r in range(NLANE):
            cols = jnp.full((NLANE,), base + r, jnp.int32)
            off = (base + r) * D
            for c in range(NV):
                vals = plsc.load_gather(blk, [c * NLANE + iota, cols])
                outst[pl.ds(off + c * NLANE, NLANE)] = vals
        return carry

    lax.fori_loop(0, width // NLANE, body, 0)


def _untile_body(tabT_hbm, tail_hbm, out_hbm, blkA, blkB, outA, outB,
                 tailbuf, semI, semO):
    cid = lax.axis_index("c")
    sid = lax.axis_index("s")
    wid = sid * NC + cid

    def fire_in(slot, buf):
        blk = wid + slot * NW

        @pl.when(blk < NBF)
        def _():
            pltpu.async_copy(tabT_hbm.at[:, pl.ds(blk * CW, CW)],
                             buf.at[:, pl.ds(0, CW)], semI)

    def wait_in(slot, buf):
        blk = wid + slot * NW

        @pl.when(blk < NBF)
        def _():
            pltpu.make_async_copy(
                tabT_hbm.at[:, pl.ds(blk * CW, CW)],
                buf.at[:, pl.ds(0, CW)], semI).wait()

    def fire_out(slot, outst):
        blk = wid + slot * NW

        @pl.when(blk < NBF)
        def _():
            pltpu.async_copy(
                outst, out_hbm.at[pl.ds(blk * (CW * D), CW * D)], semO)

    def wait_out(slot, outst):
        blk = wid + slot * NW

        @pl.when(blk < NBF)
        def _():
            pltpu.make_async_copy(
                outst, out_hbm.at[pl.ds(blk * (CW * D), CW * D)], semO).wait()

    bufs = ((blkA, outA), (blkB, outB))
    fire_in(0, blkA)

    def body(i2, carry):
        for p in range(2):
            slot = i2 * 2 + p
            buf, outst = bufs[p]
            nbuf, _ = bufs[1 - p]
            fire_in(slot + 1, nbuf)
            @pl.when(slot >= 2)
            def _():
                wait_out(slot - 2, outst)
            wait_in(slot, buf)

            if not _SKIP_COMPUTE:
                @pl.when(wid + slot * NW < NBF)
                def _():
                    _untile_compute(buf, outst, CW)

            fire_out(slot, outst)
        return carry

    lax.fori_loop(0, NSLOT // 2, body, 0)
    wait_out(NSLOT - 2, bufs[(NSLOT - 2) % 2][1])
    wait_out(NSLOT - 1, bufs[(NSLOT - 1) % 2][1])

    @pl.when(wid == 0)
    def _():
        pltpu.sync_copy(tail_hbm, tailbuf)
        _untile_compute(tailbuf, outA, VTAIL)
        pltpu.sync_copy(outA.at[pl.ds(0, VTAIL * D)],
                        out_hbm.at[pl.ds(NBF * (CW * D), VTAIL * D)])


def _untile(tabT):
    mesh = plsc.VectorSubcoreMesh(core_axis_name="c", subcore_axis_name="s")
    fn = pl.kernel(
        _untile_body,
        mesh=mesh,
        out_type=jax.ShapeDtypeStruct((V * D,), jnp.float32),
        scratch_types=[
            pltpu.VMEM((D, CW + 1), jnp.float32),
            pltpu.VMEM((D, CW + 1), jnp.float32),
            pltpu.VMEM((CW * D,), jnp.float32),
            pltpu.VMEM((CW * D,), jnp.float32),
            pltpu.VMEM((D, VTAIL), jnp.float32),
            pltpu.SemaphoreType.DMA,
            pltpu.SemaphoreType.DMA,
        ],
        compiler_params=pltpu.CompilerParams(
            needs_layout_passes=False,
            disable_bounds_checks=True,
        ),
    )
    tail = lax.slice(tabT, (0, NBF * CW), (D, V))
    return fn(tabT, tail)


def _flatten_row(src2d, dst1d, row, dst_base, n, iota):
    full = n - n % NLANE
    for c in range(0, full, NLANE):
        vals = src2d[row, pl.ds(c, NLANE)]
        plsc.store_scatter(dst1d, [dst_base + c + iota], vals)
    if full < n:
        tail = n - NLANE
        rows = jnp.full((NLANE,), row, jnp.int32)
        vals = plsc.load_gather(src2d, [rows, tail + iota])
        plsc.store_scatter(dst1d, [dst_base + tail + iota], vals)


def _flatten_body(title_hbm, snip_hbm, tout_hbm, sout_hbm,
                  t2d, s2d, tfl, sfl):
    cid = lax.axis_index("c")
    sid = lax.axis_index("s")
    wid = sid * NC + cid
    base = wid * BPW
    iota = lax.iota(jnp.int32, NLANE)

    pltpu.sync_copy(title_hbm.at[pl.ds(base, BPW)], t2d)
    pltpu.sync_copy(snip_hbm.at[pl.ds(base, BPW)], s2d)

    def body(r, carry):
        _flatten_row(t2d, tfl, r, r * TLEN, TLEN, iota)
        _flatten_row(s2d, sfl, r, r * SLEN, SLEN, iota)
        return carry

    lax.fori_loop(0, BPW, body, 0)

    pltpu.sync_copy(tfl, tout_hbm.at[pl.ds(base * TLEN, BPW * TLEN)])
    pltpu.sync_copy(sfl, sout_hbm.at[pl.ds(base * SLEN, BPW * SLEN)])


def _flatten(title, snippet):
    mesh = plsc.VectorSubcoreMesh(core_axis_name="c", subcore_axis_name="s")
    fn = pl.kernel(
        _flatten_body,
        mesh=mesh,
        out_type=(jax.ShapeDtypeStruct((B * TLEN,), jnp.int32),
                  jax.ShapeDtypeStruct((B * SLEN,), jnp.int32)),
        scratch_types=[
            pltpu.VMEM((BPW, TLEN), jnp.int32),
            pltpu.VMEM((BPW, SLEN), jnp.int32),
            pltpu.VMEM((BPW * TLEN,), jnp.int32),
            pltpu.VMEM((BPW * SLEN,), jnp.int32),
        ],
        compiler_params=pltpu.CompilerParams(needs_layout_passes=False),
    )
    return fn(title, snippet)


GROUP = 2
NG = BPW // GROUP
TG = GROUP * TLEN
SG = GROUP * SLEN


def _pool_body(tflat_hbm, sflat_hbm, ttab_hbm, stab_hbm, out_hbm,
               idx_t, idx_s, tbufA, sbufA, tbufB, sbufB, acc, semA, semB):
    cid = lax.axis_index("c")
    sid = lax.axis_index("s")
    wid = sid * NC + cid
    base = wid * BPW

    pltpu.sync_copy(tflat_hbm.at[pl.ds(base * TLEN, BPW * TLEN)], idx_t)
    pltpu.sync_copy(sflat_hbm.at[pl.ds(base * SLEN, BPW * SLEN)], idx_s)

    def fire(g, tbuf, sbuf, sem):
        pltpu.async_copy(ttab_hbm.at[idx_t.at[pl.ds(g * TG, TG)]], tbuf, sem)
        pltpu.async_copy(stab_hbm.at[idx_s.at[pl.ds(g * SG, SG)]], sbuf, sem)

    def drain(g, tbuf, sbuf, sem):
        pltpu.make_async_copy(
            ttab_hbm.at[idx_t.at[pl.ds(g * TG, TG)]], tbuf, sem).wait()
        pltpu.make_async_copy(
            stab_hbm.at[idx_s.at[pl.ds(g * SG, SG)]], sbuf, sem).wait()

    def accumulate(g, tbuf, sbuf):
        zeros = tuple(jnp.zeros((NLANE,), jnp.float32) for _ in range(NV))
        for k in range(GROUP):
            row = g * GROUP + k

            def tbody(t, vs):
                b = k * TLEN + 2 * t
                return tuple(
                    vs[c] + tbuf[b, pl.ds(c * NLANE, NLANE)]
                    + tbuf[b + 1, pl.ds(c * NLANE, NLANE)]
                    for c in range(NV))

            tv = lax.fori_loop(0, TLEN // 2, tbody, zeros)
            for c in range(NV):
                acc[row, pl.ds(c * NLANE, NLANE)] = tv[c] * (1.0 / TLEN)

            def sbody(t, vs):
                b = k * SLEN + 2 * t
                return tuple(
                    vs[c] + sbuf[b, pl.ds(c * NLANE, NLANE)]
                    + sbuf[b + 1, pl.ds(c * NLANE, NLANE)]
                    for c in range(NV))

            sv = lax.fori_loop(0, SLEN // 2, sbody, zeros)
            for c in range(NV):
                acc[row, pl.ds(D + c * NLANE, NLANE)] = sv[c] * (1.0 / SLEN)

    fire(0, tbufA, sbufA, semA)
    bufs = ((tbufA, sbufA, semA), (tbufB, sbufB, semB))

    def body(i, carry):
        for p in range(2):
            g = i * 2 + p
            tbuf, sbuf, sem = bufs[p]
            ntbuf, nsbuf, nsem = bufs[1 - p]

            @pl.when(g + 1 < NG)
            def _():
                fire(g + 1, ntbuf, nsbuf, nsem)

            drain(g, tbuf, sbuf, sem)
            accumulate(g, tbuf, sbuf)
        return carry

    lax.fori_loop(0, NG // 2, body, 0)

    pltpu.sync_copy(acc, out_hbm.at[pl.ds(base, BPW)])


def _pool(tflat, sflat, ttab, stab):
    mesh = plsc.VectorSubcoreMesh(core_axis_name="c", subcore_axis_name="s")
    fn = pl.kernel(
        _pool_body,
        mesh=mesh,
        out_type=jax.ShapeDtypeStruct((B, 2 * D), jnp.float32),
        scratch_types=[
            pltpu.VMEM((BPW * TLEN,), jnp.int32),
            pltpu.VMEM((BPW * SLEN,), jnp.int32),
            pltpu.VMEM((TG, D), jnp.float32),
            pltpu.VMEM((SG, D), jnp.float32),
            pltpu.VMEM((TG, D), jnp.float32),
            pltpu.VMEM((SG, D), jnp.float32),
            pltpu.VMEM((BPW, 2 * D), jnp.float32),
            pltpu.SemaphoreType.DMA,
            pltpu.SemaphoreType.DMA,
        ],
        compiler_params=pltpu.CompilerParams(use_tc_tiling_on_sc=False),
    )
    return fn(tflat, sflat, ttab, stab)


def _mlp_body(x_ref, w1_ref, b1_ref, w2_ref, b2_ref, o_ref):
    h = jnp.dot(x_ref[...], w1_ref[...], preferred_element_type=jnp.float32)
    h = jnp.maximum(h + b1_ref[...], 0.0)
    o_ref[...] = (jnp.dot(h, w2_ref[...], preferred_element_type=jnp.float32)
                  + b2_ref[...])


def _mlp(x, W1, b1, W2, b2):
    TB = 512
    grid = (B // TB,)
    return pl.pallas_call(
        _mlp_body,
        grid=grid,
        in_specs=[
            pl.BlockSpec((TB, 2 * D), lambda i: (i, 0)),
            pl.BlockSpec((2 * D, HID), lambda i: (0, 0)),
            pl.BlockSpec((1, HID), lambda i: (0, 0)),
            pl.BlockSpec((HID, OUT), lambda i: (0, 0)),
            pl.BlockSpec((1, OUT), lambda i: (0, 0)),
        ],
        out_specs=pl.BlockSpec((TB, OUT), lambda i: (i, 0)),
        out_shape=jax.ShapeDtypeStruct((B, OUT), jnp.float32),
    )(x, W1, b1, W2, b2)


def kernel(title, snippet, title_table, snippet_table, W1, b1, W2, b2):
    ttab = jnp.reshape(_untile(title_table.T), (V, D))
    stab = jnp.reshape(_untile(snippet_table.T), (V, D))
    tflat, sflat = _flatten(title.astype(jnp.int32), snippet.astype(jnp.int32))
    combined = _pool(tflat, sflat, ttab, stab)
    return _mlp(combined, W1, b1.reshape(1, HID), W2, b2.reshape(1, OUT))


# speedup vs baseline: 2.6157x; 2.6157x over previous
import jax
import jax.numpy as jnp
from jax import lax
from jax.experimental import pallas as pl
from jax.experimental.pallas import tpu as pltpu
from jax.experimental.pallas import tpu_sc as plsc

V = 1000000
D = 64
HID = 600
OUT = 1000
B = 4096
TLEN = 20
SLEN = 200

NC = 2
NS = 16
NW = NC * NS
BPW = B // NW
NLANE = 16
NV = D // NLANE

def _flatten_row(src2d, dst1d, row, dst_base, n, iota):
    full = n - n % NLANE
    for c in range(0, full, NLANE):
        vals = src2d[row, pl.ds(c, NLANE)]
        plsc.store_scatter(dst1d, [dst_base + c + iota], vals)
    if full < n:
        tail = n - NLANE
        rows = jnp.full((NLANE,), row, jnp.int32)
        vals = plsc.load_gather(src2d, [rows, tail + iota])
        plsc.store_scatter(dst1d, [dst_base + tail + iota], vals)


def _flatten_body(title_hbm, snip_hbm, tout_hbm, sout_hbm,
                  t2d, s2d, tfl, sfl):
    cid = lax.axis_index("c")
    sid = lax.axis_index("s")
    wid = sid * NC + cid
    base = wid * BPW
    iota = lax.iota(jnp.int32, NLANE)

    pltpu.sync_copy(title_hbm.at[pl.ds(base, BPW)], t2d)
    pltpu.sync_copy(snip_hbm.at[pl.ds(base, BPW)], s2d)

    def body(r, carry):
        _flatten_row(t2d, tfl, r, r * TLEN, TLEN, iota)
        _flatten_row(s2d, sfl, r, r * SLEN, SLEN, iota)
        return carry

    lax.fori_loop(0, BPW, body, 0)

    pltpu.sync_copy(tfl, tout_hbm.at[pl.ds(base * TLEN, BPW * TLEN)])
    pltpu.sync_copy(sfl, sout_hbm.at[pl.ds(base * SLEN, BPW * SLEN)])


def _flatten(title, snippet):
    mesh = plsc.VectorSubcoreMesh(core_axis_name="c", subcore_axis_name="s")
    fn = pl.kernel(
        _flatten_body,
        mesh=mesh,
        out_type=(jax.ShapeDtypeStruct((B * TLEN,), jnp.int32),
                  jax.ShapeDtypeStruct((B * SLEN,), jnp.int32)),
        scratch_types=[
            pltpu.VMEM((BPW, TLEN), jnp.int32),
            pltpu.VMEM((BPW, SLEN), jnp.int32),
            pltpu.VMEM((BPW * TLEN,), jnp.int32),
            pltpu.VMEM((BPW * SLEN,), jnp.int32),
        ],
        compiler_params=pltpu.CompilerParams(needs_layout_passes=False),
    )
    return fn(title, snippet)


GROUP = 2
NG = BPW // GROUP
TG = GROUP * TLEN
SG = GROUP * SLEN


def _pool_body(tflat_hbm, sflat_hbm, ttab_hbm, stab_hbm, out_hbm,
               idx_t, idx_s, tbufA, sbufA, tbufB, sbufB, acc, semA, semB):
    cid = lax.axis_index("c")
    sid = lax.axis_index("s")
    wid = sid * NC + cid
    base = wid * BPW

    pltpu.sync_copy(tflat_hbm.at[pl.ds(base * TLEN, BPW * TLEN)], idx_t)
    pltpu.sync_copy(sflat_hbm.at[pl.ds(base * SLEN, BPW * SLEN)], idx_s)

    def fire(g, tbuf, sbuf, sem):
        pltpu.async_copy(ttab_hbm.at[idx_t.at[pl.ds(g * TG, TG)]], tbuf, sem)
        pltpu.async_copy(stab_hbm.at[idx_s.at[pl.ds(g * SG, SG)]], sbuf, sem)

    def drain(g, tbuf, sbuf, sem):
        pltpu.make_async_copy(
            ttab_hbm.at[idx_t.at[pl.ds(g * TG, TG)]], tbuf, sem).wait()
        pltpu.make_async_copy(
            stab_hbm.at[idx_s.at[pl.ds(g * SG, SG)]], sbuf, sem).wait()

    def accumulate(g, tbuf, sbuf):
        zeros = tuple(jnp.zeros((NLANE,), jnp.float32) for _ in range(NV))
        for k in range(GROUP):
            row = g * GROUP + k

            def tbody(t, vs):
                b = k * TLEN + 2 * t
                return tuple(
                    vs[c] + tbuf[b, pl.ds(c * NLANE, NLANE)]
                    + tbuf[b + 1, pl.ds(c * NLANE, NLANE)]
                    for c in range(NV))

            tv = lax.fori_loop(0, TLEN // 2, tbody, zeros)
            for c in range(NV):
                acc[row, pl.ds(c * NLANE, NLANE)] = tv[c] * (1.0 / TLEN)

            def sbody(t, vs):
                b = k * SLEN + 2 * t
                return tuple(
                    vs[c] + sbuf[b, pl.ds(c * NLANE, NLANE)]
                    + sbuf[b + 1, pl.ds(c * NLANE, NLANE)]
                    for c in range(NV))

            sv = lax.fori_loop(0, SLEN // 2, sbody, zeros)
            for c in range(NV):
                acc[row, pl.ds(D + c * NLANE, NLANE)] = sv[c] * (1.0 / SLEN)

    fire(0, tbufA, sbufA, semA)
    bufs = ((tbufA, sbufA, semA), (tbufB, sbufB, semB))

    def body(i, carry):
        for p in range(2):
            g = i * 2 + p
            tbuf, sbuf, sem = bufs[p]
            ntbuf, nsbuf, nsem = bufs[1 - p]

            @pl.when(g + 1 < NG)
            def _():
                fire(g + 1, ntbuf, nsbuf, nsem)

            drain(g, tbuf, sbuf, sem)
            accumulate(g, tbuf, sbuf)
        return carry

    lax.fori_loop(0, NG // 2, body, 0)

    pltpu.sync_copy(acc, out_hbm.at[pl.ds(base, BPW)])


def _pool(tflat, sflat, ttab, stab):
    mesh = plsc.VectorSubcoreMesh(core_axis_name="c", subcore_axis_name="s")
    fn = pl.kernel(
        _pool_body,
        mesh=mesh,
        out_type=jax.ShapeDtypeStruct((B, 2 * D), jnp.float32),
        scratch_types=[
            pltpu.VMEM((BPW * TLEN,), jnp.int32),
            pltpu.VMEM((BPW * SLEN,), jnp.int32),
            pltpu.VMEM((TG, D), jnp.float32),
            pltpu.VMEM((SG, D), jnp.float32),
            pltpu.VMEM((TG, D), jnp.float32),
            pltpu.VMEM((SG, D), jnp.float32),
            pltpu.VMEM((BPW, 2 * D), jnp.float32),
            pltpu.SemaphoreType.DMA,
            pltpu.SemaphoreType.DMA,
        ],
        compiler_params=pltpu.CompilerParams(use_tc_tiling_on_sc=False),
    )
    return fn(tflat, sflat, ttab, stab)


def _mlp_body(x_ref, w1_ref, b1_ref, w2_ref, b2_ref, o_ref):
    h = jnp.dot(x_ref[...], w1_ref[...], preferred_element_type=jnp.float32)
    h = jnp.maximum(h + b1_ref[...], 0.0)
    o_ref[...] = (jnp.dot(h, w2_ref[...], preferred_element_type=jnp.float32)
                  + b2_ref[...])


def _mlp(x, W1, b1, W2, b2):
    TB = 512
    grid = (B // TB,)
    return pl.pallas_call(
        _mlp_body,
        grid=grid,
        in_specs=[
            pl.BlockSpec((TB, 2 * D), lambda i: (i, 0)),
            pl.BlockSpec((2 * D, HID), lambda i: (0, 0)),
            pl.BlockSpec((1, HID), lambda i: (0, 0)),
            pl.BlockSpec((HID, OUT), lambda i: (0, 0)),
            pl.BlockSpec((1, OUT), lambda i: (0, 0)),
        ],
        out_specs=pl.BlockSpec((TB, OUT), lambda i: (i, 0)),
        out_shape=jax.ShapeDtypeStruct((B, OUT), jnp.float32),
    )(x, W1, b1, W2, b2)


def kernel(title, snippet, title_table, snippet_table, W1, b1, W2, b2):
    tflat, sflat = _flatten(title.astype(jnp.int32), snippet.astype(jnp.int32))
    combined = _pool(tflat, sflat, title_table, snippet_table)
    return _mlp(combined, W1, b1.reshape(1, HID), W2, b2.reshape(1, OUT))
